# baseline (speedup 1.0000x reference)
import functools

import jax
import jax.numpy as jnp
from jax import lax
from jax.experimental import pallas as pl
from jax.experimental.pallas import tpu as pltpu
from jax.experimental.pallas import tpu_sc as plsc

MAX_REL = 512
NUM_HEADS = 16
SEQ_LEN = 2048
TABLE_ROWS = 2 * MAX_REL + 1
BROWS = 128
NROW = SEQ_LEN // BROWS
ESW = (SEQ_LEN - BROWS) + SEQ_LEN
BLEN = 4096

_NC, _NS = 2, 16
_NW = _NC * _NS
_CPW = (NUM_HEADS * BLEN) // _NW


@functools.partial(
    pl.kernel,
    out_type=jax.ShapeDtypeStruct((NUM_HEADS * BLEN,), jnp.float32),
    mesh=plsc.VectorSubcoreMesh(core_axis_name="c", subcore_axis_name="s"),
    compiler_params=pltpu.CompilerParams(needs_layout_passes=False),
    scratch_types=[
        pltpu.VMEM((NUM_HEADS * TABLE_ROWS,), jnp.float32),
        pltpu.VMEM((_CPW,), jnp.float32),
    ],
)
def _diag_gather(tab_hbm, b_hbm, tab_v, out_v):
    wid = lax.axis_index("s") * _NC + lax.axis_index("c")
    h = wid // 2
    lane0 = (wid % 2) * _CPW
    pltpu.sync_copy(tab_hbm, tab_v)
    row0 = jnp.full((16,), h * TABLE_ROWS, dtype=jnp.int32)
    lane = jnp.arange(16, dtype=jnp.int32)

    def body(j, carry):
        c = lane0 + j * 16 + lane
        idx = jnp.clip(c - (SEQ_LEN - 1), -MAX_REL, MAX_REL) + MAX_REL
        out_v[pl.ds(j * 16, 16)] = plsc.load_gather(tab_v, [row0 + idx])
        return carry

    lax.fori_loop(0, _CPW // 16, body, 0)
    pltpu.sync_copy(out_v, b_hbm.at[pl.ds(wid * _CPW, _CPW)])


def _toeplitz_body(b_ref, out_ref, es_ref):
    ib = pl.program_id(1)

    @pl.when(ib == 0)
    def _build():
        for r in range(BROWS):
            es_ref[r, :] = b_ref[0, 0, pl.ds(BROWS - 1 - r, ESW)]

    off = pl.multiple_of((NROW - 1 - ib) * BROWS, BROWS)
    out_ref[0] = es_ref[:, pl.ds(off, SEQ_LEN)]


def kernel(table, seq_len):
    del seq_len
    b = _diag_gather(table.T.reshape(-1))
    b = b.reshape(NUM_HEADS, 1, BLEN)

    return pl.pallas_call(
        _toeplitz_body,
        grid=(NUM_HEADS, NROW),
        in_specs=[pl.BlockSpec((1, 1, BLEN), lambda h, ib: (h, 0, 0))],
        out_specs=pl.BlockSpec((1, BROWS, SEQ_LEN), lambda h, ib: (h, ib, 0)),
        out_shape=jax.ShapeDtypeStruct((NUM_HEADS, SEQ_LEN, SEQ_LEN), jnp.float32),
        scratch_shapes=[pltpu.VMEM((BROWS, ESW), jnp.float32)],
    )(b)


# speedup vs baseline: 1.0033x; 1.0033x over previous
import functools

import jax
import jax.numpy as jnp
from jax import lax
from jax.experimental import pallas as pl
from jax.experimental.pallas import tpu as pltpu
from jax.experimental.pallas import tpu_sc as plsc

MAX_REL = 512
NUM_HEADS = 16
SEQ_LEN = 2048
TABLE_ROWS = 2 * MAX_REL + 1
BROWS = 128
NROW = SEQ_LEN // BROWS
ESW = (SEQ_LEN - BROWS) + SEQ_LEN
BLEN = 4096

_NC, _NS = 2, 16
_NW = _NC * _NS
_CPW = (NUM_HEADS * BLEN) // _NW


@functools.partial(
    pl.kernel,
    out_type=jax.ShapeDtypeStruct((NUM_HEADS * BLEN,), jnp.float32),
    mesh=plsc.VectorSubcoreMesh(core_axis_name="c", subcore_axis_name="s"),
    compiler_params=pltpu.CompilerParams(needs_layout_passes=False),
    scratch_types=[
        pltpu.VMEM((TABLE_ROWS * NUM_HEADS,), jnp.float32),
        pltpu.VMEM((_CPW,), jnp.float32),
    ],
)
def _diag_gather(tab_hbm, b_hbm, tab_v, out_v):
    wid = lax.axis_index("s") * _NC + lax.axis_index("c")
    h = wid // 2
    lane0 = (wid % 2) * _CPW
    pltpu.sync_copy(tab_hbm, tab_v)
    head = jnp.full((16,), h, dtype=jnp.int32)
    lane = jnp.arange(16, dtype=jnp.int32)

    def body(j, carry):
        c = lane0 + j * 16 + lane
        idx = jnp.clip(c - (SEQ_LEN - 1), -MAX_REL, MAX_REL) + MAX_REL
        out_v[pl.ds(j * 16, 16)] = plsc.load_gather(
            tab_v, [idx * NUM_HEADS + head])
        return carry

    lax.fori_loop(0, _CPW // 16, body, 0)
    pltpu.sync_copy(out_v, b_hbm.at[pl.ds(wid * _CPW, _CPW)])


def _toeplitz_body(b_ref, out_ref, es_ref):
    ib = pl.program_id(1)

    @pl.when(ib == 0)
    def _build():
        for r in range(BROWS):
            es_ref[r, :] = b_ref[0, 0, pl.ds(BROWS - 1 - r, ESW)]

    off = pl.multiple_of((NROW - 1 - ib) * BROWS, BROWS)
    out_ref[0] = es_ref[:, pl.ds(off, SEQ_LEN)]


def kernel(table, seq_len):
    del seq_len
    b = _diag_gather(table.reshape(-1))
    b = b.reshape(NUM_HEADS, 1, BLEN)

    return pl.pallas_call(
        _toeplitz_body,
        grid=(NUM_HEADS, NROW),
        in_specs=[pl.BlockSpec((1, 1, BLEN), lambda h, ib: (h, 0, 0))],
        out_specs=pl.BlockSpec((1, BROWS, SEQ_LEN), lambda h, ib: (h, ib, 0)),
        out_shape=jax.ShapeDtypeStruct((NUM_HEADS, SEQ_LEN, SEQ_LEN), jnp.float32),
        scratch_shapes=[pltpu.VMEM((BROWS, ESW), jnp.float32)],
    )(b)
